# baseline (speedup 1.0000x reference)
import jax
import jax.numpy as jnp
from jax import lax
from jax.experimental import pallas as pl
from jax.experimental.pallas import tpu as pltpu
from jax.experimental.pallas import tpu_sc as plsc

NUM_EMB = 1000000
DIM = 64
PROWS = NUM_EMB // 2
NSEQ = 16384
SEQ = 50
NW = 32
SLAB = NSEQ // NW
SB = 128
BPS = SLAB // SB
NBLK = SEQ * BPS


CH = 192
NFULL = PROWS // CH
MAXTRIP = (NFULL + NW - 1) // NW
TAILROWS = 96
TAILP0 = PROWS - TAILROWS


def _repack_body(tT_hbm, packed_hbm, in0, out0, gs0, os0, in1, out1, gs1, os1,
                 in_t):
    nc = 2
    wid = lax.axis_index("s") * nc + lax.axis_index("c")

    def chunk_of(t):
        return wid + t * NW

    def prep(c, in_v, gsem):
        @pl.when(c < NFULL)
        def _():
            for h in range(4):
                pltpu.async_copy(
                    tT_hbm.at[pl.ds(16 * h, 16), pl.ds(c * (2 * CH), 2 * CH)],
                    in_v.at[pl.ds(16 * h, 16), :], gsem)

    def transpose_into(out_v, in_v, nrows):
        i16 = lax.iota(jnp.int32, 16)
        i2 = 2 * i16

        @plsc.parallel_loop(0, nrows // 16, unroll=1)
        def _rb(rb):
            rvec = i16 + rb * 16
            for cb in range(8):
                cols_in = (2 * 16) * rb + (cb >> 2) + i2
                rbase = (cb & 3) * 16
                for k in range(16):
                    dd = lax.bitwise_and(i16 + k, 15)
                    vals = plsc.load_gather(in_v, [rbase + dd, cols_in])
                    plsc.store_scatter(out_v, [rvec, cb * 16 + dd], vals)

    def finish(c, in_v, out_v, gsem, osem):
        @pl.when(c < NFULL)
        def _():
            pltpu.make_async_copy(
                out_v, packed_hbm.at[pl.ds(0, CH), :], osem).wait()
            for h in range(4):
                pltpu.make_async_copy(
                    tT_hbm.at[pl.ds(16 * h, 16), pl.ds(c * (2 * CH), 2 * CH)],
                    in_v.at[pl.ds(16 * h, 16), :], gsem).wait()
            transpose_into(out_v, in_v, CH)
            pltpu.async_copy(out_v, packed_hbm.at[pl.ds(c * CH, CH), :], osem)

    prep(chunk_of(0), in0, gs0)
    prep(chunk_of(1), in1, gs1)
    pltpu.async_copy(out0, packed_hbm.at[pl.ds(chunk_of(0) * CH, CH), :], os0)
    pltpu.async_copy(out1, packed_hbm.at[pl.ds(chunk_of(1) * CH, CH), :], os1)

    @pl.loop(0, MAXTRIP, step=2)
    def _pair(t):
        finish(chunk_of(t), in0, out0, gs0, os0)
        prep(chunk_of(t + 2), in0, gs0)
        finish(chunk_of(t + 1), in1, out1, gs1, os1)
        prep(chunk_of(t + 3), in1, gs1)

    @pl.when(chunk_of(MAXTRIP - 2) < NFULL)
    def _d0():
        pltpu.make_async_copy(out0, packed_hbm.at[pl.ds(0, CH), :], os0).wait()

    @pl.when(chunk_of(MAXTRIP - 1) < NFULL)
    def _d1():
        pltpu.make_async_copy(out1, packed_hbm.at[pl.ds(0, CH), :], os1).wait()

    def transpose_small(out_v, in_v, nrows):
        i16 = lax.iota(jnp.int32, 16)
        i2 = 2 * i16

        @pl.loop(0, nrows // 16)
        def _rb(rb):
            rvec = i16 + rb * 16

            @pl.loop(0, 8)
            def _cb(cb):
                cols_in = (2 * 16) * rb + lax.shift_right_logical(cb, 2) + i2
                rbase = lax.bitwise_and(cb, 3) * 16
                for k in range(16):
                    dd = lax.bitwise_and(i16 + k, 15)
                    vals = plsc.load_gather(in_v, [rbase + dd, cols_in])
                    plsc.store_scatter(out_v, [rvec, cb * 16 + dd], vals)

    @pl.when(wid == 0)
    def _tail():
        pltpu.sync_copy(tT_hbm.at[:, pl.ds(2 * TAILP0, 2 * TAILROWS)], in_t)
        transpose_small(out0, in_t, TAILROWS)
        pltpu.sync_copy(out0.at[pl.ds(0, TAILROWS), :],
                        packed_hbm.at[pl.ds(TAILP0, TAILROWS), :])


@jax.jit
def _repack(tT):
    mesh = plsc.VectorSubcoreMesh(core_axis_name="c", subcore_axis_name="s")
    return pl.kernel(
        _repack_body,
        out_type=jax.ShapeDtypeStruct((PROWS, 128), jnp.float32),
        mesh=mesh,
        scratch_types=[
            pltpu.VMEM((DIM, 2 * CH), jnp.float32),
            pltpu.VMEM((CH, 128), jnp.float32),
            pltpu.SemaphoreType.DMA,
            pltpu.SemaphoreType.DMA,
            pltpu.VMEM((DIM, 2 * CH), jnp.float32),
            pltpu.VMEM((CH, 128), jnp.float32),
            pltpu.SemaphoreType.DMA,
            pltpu.SemaphoreType.DMA,
            pltpu.VMEM((DIM, 2 * TAILROWS), jnp.float32),
        ],
        compiler_params=pltpu.CompilerParams(
            use_tc_tiling_on_sc=True, needs_layout_passes=False),
    )(tT)


def _body(idxT_hbm, packed_hbm, outT_hbm, idx_v,
          qv0, jv0, buf0, oblk0, gsem0,
          qv1, jv1, buf1, oblk1, gsem1):
    nc = 2
    wid = lax.axis_index("s") * nc + lax.axis_index("c")
    s0 = wid * SLAB
    pltpu.sync_copy(idxT_hbm.at[:, pl.ds(s0, SLAB)], idx_v)

    def prep_start(b, qv, jv, buf, gsem):
        p = lax.div(b, BPS)
        sb = lax.rem(b, BPS)

        @pl.loop(0, SB // 16)
        def _q(k):
            v = idx_v[p, pl.ds(sb * SB + k * 16, 16)]
            qv[pl.ds(k * 16, 16)] = lax.shift_right_logical(v, 1)
            jv[pl.ds(k * 16, 16)] = lax.bitwise_and(v, 1)

        pltpu.async_copy(packed_hbm.at[qv], buf, gsem)

    def finish(b, qv, jv, buf, oblk, gsem):
        p = lax.div(b, BPS)
        sb = lax.rem(b, BPS)
        pltpu.make_async_copy(packed_hbm.at[qv], buf, gsem).wait()

        @plsc.parallel_loop(0, SB // 16, unroll=4)
        def _tb(tb):
            i16 = lax.iota(jnp.int32, 16)
            trow = i16 + tb * 16
            cbase = jv[pl.ds(tb * 16, 16)] * 64
            for db in range(DIM // 16):
                cb = cbase + db * 16
                for k in range(16):
                    dd = lax.bitwise_and(i16 + k, 15)
                    vals = plsc.load_gather(buf, [trow, cb + dd])
                    plsc.store_scatter(oblk, [db * 16 + dd, trow], vals)

        pltpu.sync_copy(oblk, outT_hbm.at[p, :, pl.ds(s0 + sb * SB, SB)])

    prep_start(0, qv0, jv0, buf0, gsem0)

    @pl.loop(0, NBLK, step=2)
    def _pair(g):
        prep_start(g + 1, qv1, jv1, buf1, gsem1)
        finish(g, qv0, jv0, buf0, oblk0, gsem0)

        @pl.when(g + 2 < NBLK)
        def _pre():
            prep_start(g + 2, qv0, jv0, buf0, gsem0)

        finish(g + 1, qv1, jv1, buf1, oblk1, gsem1)


@jax.jit
def _gather2(idxT, packed):
    mesh = plsc.VectorSubcoreMesh(core_axis_name="c", subcore_axis_name="s")
    return pl.kernel(
        _body,
        out_type=jax.ShapeDtypeStruct((SEQ, DIM, NSEQ), jnp.float32),
        mesh=mesh,
        scratch_types=[
            pltpu.VMEM((SEQ, SLAB), jnp.int32),
        ] + 2 * [
            pltpu.VMEM((SB,), jnp.int32),
            pltpu.VMEM((SB,), jnp.int32),
            pltpu.VMEM((SB, 128), jnp.float32),
            pltpu.VMEM((DIM, SB), jnp.float32),
            pltpu.SemaphoreType.DMA,
        ],
        compiler_params=pltpu.CompilerParams(
            use_tc_tiling_on_sc=True, needs_layout_passes=False),
    )(idxT, packed)


def kernel(indices, table):
    packed = _repack(table.T)
    outT = _gather2(indices.T, packed)
    return jnp.transpose(outT, (2, 0, 1))


# speedup vs baseline: 1.2490x; 1.2490x over previous
import jax
import jax.numpy as jnp
from jax import lax
from jax.experimental import pallas as pl
from jax.experimental.pallas import tpu as pltpu
from jax.experimental.pallas import tpu_sc as plsc

NUM_EMB = 1000000
DIM = 64
PROWS = NUM_EMB // 2
NSEQ = 16384
SEQ = 50
NW = 32
SLAB = NSEQ // NW
SB = 128
BPS = SLAB // SB
NBLK = SEQ * BPS


CH = 192
NFULL = PROWS // CH
MAXTRIP = (NFULL + NW - 1) // NW
TAILROWS = 96
TAILP0 = PROWS - TAILROWS


def _repack_body(tT_hbm, packed_hbm, in0, out0, gs0, os0, in1, out1, gs1, os1,
                 in_t):
    nc = 2
    wid = lax.axis_index("s") * nc + lax.axis_index("c")

    def chunk_of(t):
        return wid + t * NW

    def prep(c, in_v, gsem):
        @pl.when(c < NFULL)
        def _():
            for h in range(4):
                pltpu.async_copy(
                    tT_hbm.at[pl.ds(16 * h, 16), pl.ds(c * (2 * CH), 2 * CH)],
                    in_v.at[pl.ds(16 * h, 16), :], gsem)

    def transpose_into(out_v, in_v, nrows):
        i16 = lax.iota(jnp.int32, 16)
        ish = lax.shift_right_logical(i16, 1)
        i64 = 64 * lax.bitwise_and(i16, 1)

        @plsc.parallel_loop(0, nrows // 8, unroll=2)
        def _rb(rb8):
            cin = rb8 * 16 + i16
            rows_out = rb8 * 8 + ish
            for db in range(4):
                for k in range(16):
                    dd = lax.bitwise_and(i16 + k, 15)
                    vals = plsc.load_gather(in_v, [db * 16 + dd, cin])
                    plsc.store_scatter(
                        out_v, [rows_out, i64 + db * 16 + dd], vals)

    def finish(c, in_v, out_v, gsem, osem):
        @pl.when(c < NFULL)
        def _():
            pltpu.make_async_copy(
                out_v, packed_hbm.at[pl.ds(0, CH), :], osem).wait()
            for h in range(4):
                pltpu.make_async_copy(
                    tT_hbm.at[pl.ds(16 * h, 16), pl.ds(c * (2 * CH), 2 * CH)],
                    in_v.at[pl.ds(16 * h, 16), :], gsem).wait()
            transpose_into(out_v, in_v, CH)
            pltpu.async_copy(out_v, packed_hbm.at[pl.ds(c * CH, CH), :], osem)

    prep(chunk_of(0), in0, gs0)
    prep(chunk_of(1), in1, gs1)
    pltpu.async_copy(out0, packed_hbm.at[pl.ds(chunk_of(0) * CH, CH), :], os0)
    pltpu.async_copy(out1, packed_hbm.at[pl.ds(chunk_of(1) * CH, CH), :], os1)

    @pl.loop(0, MAXTRIP, step=2)
    def _pair(t):
        finish(chunk_of(t), in0, out0, gs0, os0)
        prep(chunk_of(t + 2), in0, gs0)
        finish(chunk_of(t + 1), in1, out1, gs1, os1)
        prep(chunk_of(t + 3), in1, gs1)

    @pl.when(chunk_of(MAXTRIP - 2) < NFULL)
    def _d0():
        pltpu.make_async_copy(out0, packed_hbm.at[pl.ds(0, CH), :], os0).wait()

    @pl.when(chunk_of(MAXTRIP - 1) < NFULL)
    def _d1():
        pltpu.make_async_copy(out1, packed_hbm.at[pl.ds(0, CH), :], os1).wait()

    def transpose_small(out_v, in_v, nrows):
        i16 = lax.iota(jnp.int32, 16)
        i2 = 2 * i16

        @pl.loop(0, nrows // 16)
        def _rb(rb):
            rvec = i16 + rb * 16

            @pl.loop(0, 8)
            def _cb(cb):
                cols_in = (2 * 16) * rb + lax.shift_right_logical(cb, 2) + i2
                rbase = lax.bitwise_and(cb, 3) * 16
                for k in range(16):
                    dd = lax.bitwise_and(i16 + k, 15)
                    vals = plsc.load_gather(in_v, [rbase + dd, cols_in])
                    plsc.store_scatter(out_v, [rvec, cb * 16 + dd], vals)

    @pl.when(wid == 0)
    def _tail():
        pltpu.sync_copy(tT_hbm.at[:, pl.ds(2 * TAILP0, 2 * TAILROWS)], in_t)
        transpose_small(out0, in_t, TAILROWS)
        pltpu.sync_copy(out0.at[pl.ds(0, TAILROWS), :],
                        packed_hbm.at[pl.ds(TAILP0, TAILROWS), :])


@jax.jit
def _repack(tT):
    mesh = plsc.VectorSubcoreMesh(core_axis_name="c", subcore_axis_name="s")
    return pl.kernel(
        _repack_body,
        out_type=jax.ShapeDtypeStruct((PROWS, 128), jnp.float32),
        mesh=mesh,
        scratch_types=[
            pltpu.VMEM((DIM, 2 * CH), jnp.float32),
            pltpu.VMEM((CH, 128), jnp.float32),
            pltpu.SemaphoreType.DMA,
            pltpu.SemaphoreType.DMA,
            pltpu.VMEM((DIM, 2 * CH), jnp.float32),
            pltpu.VMEM((CH, 128), jnp.float32),
            pltpu.SemaphoreType.DMA,
            pltpu.SemaphoreType.DMA,
            pltpu.VMEM((DIM, 2 * TAILROWS), jnp.float32),
        ],
        compiler_params=pltpu.CompilerParams(
            use_tc_tiling_on_sc=True, needs_layout_passes=False),
    )(tT)


def _body(idxT_hbm, packed_hbm, outT_hbm, idx_v,
          qv0, jv0, buf0, oblk0, gsem0,
          qv1, jv1, buf1, oblk1, gsem1):
    nc = 2
    wid = lax.axis_index("s") * nc + lax.axis_index("c")
    s0 = wid * SLAB
    pltpu.sync_copy(idxT_hbm.at[:, pl.ds(s0, SLAB)], idx_v)

    def prep_start(b, qv, jv, buf, gsem):
        p = lax.div(b, BPS)
        sb = lax.rem(b, BPS)

        @pl.loop(0, SB // 16)
        def _q(k):
            v = idx_v[p, pl.ds(sb * SB + k * 16, 16)]
            qv[pl.ds(k * 16, 16)] = lax.shift_right_logical(v, 1)
            jv[pl.ds(k * 16, 16)] = lax.bitwise_and(v, 1)

        pltpu.async_copy(packed_hbm.at[qv], buf, gsem)

    def finish(b, qv, jv, buf, oblk, gsem):
        p = lax.div(b, BPS)
        sb = lax.rem(b, BPS)
        pltpu.make_async_copy(packed_hbm.at[qv], buf, gsem).wait()

        @plsc.parallel_loop(0, SB // 16, unroll=4)
        def _tb(tb):
            i16 = lax.iota(jnp.int32, 16)
            trow = i16 + tb * 16
            cbase = jv[pl.ds(tb * 16, 16)] * 64
            for db in range(DIM // 16):
                cb = cbase + db * 16
                for k in range(16):
                    dd = lax.bitwise_and(i16 + k, 15)
                    vals = plsc.load_gather(buf, [trow, cb + dd])
                    plsc.store_scatter(oblk, [db * 16 + dd, trow], vals)

        pltpu.sync_copy(oblk, outT_hbm.at[p, :, pl.ds(s0 + sb * SB, SB)])

    prep_start(0, qv0, jv0, buf0, gsem0)

    @pl.loop(0, NBLK, step=2)
    def _pair(g):
        prep_start(g + 1, qv1, jv1, buf1, gsem1)
        finish(g, qv0, jv0, buf0, oblk0, gsem0)

        @pl.when(g + 2 < NBLK)
        def _pre():
            prep_start(g + 2, qv0, jv0, buf0, gsem0)

        finish(g + 1, qv1, jv1, buf1, oblk1, gsem1)


@jax.jit
def _gather2(idxT, packed):
    mesh = plsc.VectorSubcoreMesh(core_axis_name="c", subcore_axis_name="s")
    return pl.kernel(
        _body,
        out_type=jax.ShapeDtypeStruct((SEQ, DIM, NSEQ), jnp.float32),
        mesh=mesh,
        scratch_types=[
            pltpu.VMEM((SEQ, SLAB), jnp.int32),
        ] + 2 * [
            pltpu.VMEM((SB,), jnp.int32),
            pltpu.VMEM((SB,), jnp.int32),
            pltpu.VMEM((SB, 128), jnp.float32),
            pltpu.VMEM((DIM, SB), jnp.float32),
            pltpu.SemaphoreType.DMA,
        ],
        compiler_params=pltpu.CompilerParams(
            use_tc_tiling_on_sc=True, needs_layout_passes=False),
    )(idxT, packed)


def kernel(indices, table):
    packed = _repack(table.T)
    outT = _gather2(indices.T, packed)
    return jnp.transpose(outT, (2, 0, 1))


# speedup vs baseline: 1.6163x; 1.2941x over previous
import jax
import jax.numpy as jnp
from jax import lax
from jax.experimental import pallas as pl
from jax.experimental.pallas import tpu as pltpu
from jax.experimental.pallas import tpu_sc as plsc

NUM_EMB = 1000000
DIM = 64
PROWS = NUM_EMB // 2
NSEQ = 16384
SEQ = 50
NW = 32
SLAB = NSEQ // NW
SB = 128
BPS = SLAB // SB
NBLK = SEQ * BPS


CH = 192
NFULL = PROWS // CH
MAXTRIP = (NFULL + NW - 1) // NW
TAILROWS = 96
TAILP0 = PROWS - TAILROWS


def _repack_body(tT_hbm, packed_hbm, in0, out0, gs0, os0, in1, out1, gs1, os1,
                 in_t):
    nc = 2
    wid = lax.axis_index("s") * nc + lax.axis_index("c")

    def chunk_of(t):
        return wid + t * NW

    def prep(c, in_v, gsem):
        @pl.when(c < NFULL)
        def _():
            for h in range(4):
                pltpu.async_copy(
                    tT_hbm.at[pl.ds(16 * h, 16), pl.ds(c * (2 * CH), 2 * CH)],
                    in_v.at[pl.ds(16 * h, 16), :], gsem)

    def transpose_into(out_v, in_v, nrows):
        i16 = lax.iota(jnp.int32, 16)
        ish = lax.shift_right_logical(i16, 1)
        i64 = 64 * lax.bitwise_and(i16, 1)

        @plsc.parallel_loop(0, nrows // 8, unroll=4)
        def _rb(rb8):
            cin = rb8 * 16 + i16
            rows_out = rb8 * 8 + ish
            for db in range(4):
                for k in range(16):
                    dd = lax.bitwise_and(i16 + k, 15)
                    vals = plsc.load_gather(in_v, [db * 16 + dd, cin])
                    plsc.store_scatter(
                        out_v, [rows_out, i64 + db * 16 + dd], vals)

    def finish(c, in_v, out_v, gsem, osem):
        @pl.when(c < NFULL)
        def _():
            pltpu.make_async_copy(
                out_v, packed_hbm.at[pl.ds(0, CH), :], osem).wait()
            for h in range(4):
                pltpu.make_async_copy(
                    tT_hbm.at[pl.ds(16 * h, 16), pl.ds(c * (2 * CH), 2 * CH)],
                    in_v.at[pl.ds(16 * h, 16), :], gsem).wait()
            transpose_into(out_v, in_v, CH)
            pltpu.async_copy(out_v, packed_hbm.at[pl.ds(c * CH, CH), :], osem)

    prep(chunk_of(0), in0, gs0)
    prep(chunk_of(1), in1, gs1)
    pltpu.async_copy(out0, packed_hbm.at[pl.ds(chunk_of(0) * CH, CH), :], os0)
    pltpu.async_copy(out1, packed_hbm.at[pl.ds(chunk_of(1) * CH, CH), :], os1)

    @pl.loop(0, MAXTRIP, step=2)
    def _pair(t):
        finish(chunk_of(t), in0, out0, gs0, os0)
        prep(chunk_of(t + 2), in0, gs0)
        finish(chunk_of(t + 1), in1, out1, gs1, os1)
        prep(chunk_of(t + 3), in1, gs1)

    @pl.when(chunk_of(MAXTRIP - 2) < NFULL)
    def _d0():
        pltpu.make_async_copy(out0, packed_hbm.at[pl.ds(0, CH), :], os0).wait()

    @pl.when(chunk_of(MAXTRIP - 1) < NFULL)
    def _d1():
        pltpu.make_async_copy(out1, packed_hbm.at[pl.ds(0, CH), :], os1).wait()

    def transpose_small(out_v, in_v, nrows):
        i16 = lax.iota(jnp.int32, 16)
        i2 = 2 * i16

        @pl.loop(0, nrows // 16)
        def _rb(rb):
            rvec = i16 + rb * 16

            @pl.loop(0, 8)
            def _cb(cb):
                cols_in = (2 * 16) * rb + lax.shift_right_logical(cb, 2) + i2
                rbase = lax.bitwise_and(cb, 3) * 16
                for k in range(16):
                    dd = lax.bitwise_and(i16 + k, 15)
                    vals = plsc.load_gather(in_v, [rbase + dd, cols_in])
                    plsc.store_scatter(out_v, [rvec, cb * 16 + dd], vals)

    @pl.when(wid == 0)
    def _tail():
        pltpu.sync_copy(tT_hbm.at[:, pl.ds(2 * TAILP0, 2 * TAILROWS)], in_t)
        transpose_small(out0, in_t, TAILROWS)
        pltpu.sync_copy(out0.at[pl.ds(0, TAILROWS), :],
                        packed_hbm.at[pl.ds(TAILP0, TAILROWS), :])


@jax.jit
def _repack(tT):
    mesh = plsc.VectorSubcoreMesh(core_axis_name="c", subcore_axis_name="s")
    return pl.kernel(
        _repack_body,
        out_type=jax.ShapeDtypeStruct((PROWS, 128), jnp.float32),
        mesh=mesh,
        scratch_types=[
            pltpu.VMEM((DIM, 2 * CH), jnp.float32),
            pltpu.VMEM((CH, 128), jnp.float32),
            pltpu.SemaphoreType.DMA,
            pltpu.SemaphoreType.DMA,
            pltpu.VMEM((DIM, 2 * CH), jnp.float32),
            pltpu.VMEM((CH, 128), jnp.float32),
            pltpu.SemaphoreType.DMA,
            pltpu.SemaphoreType.DMA,
            pltpu.VMEM((DIM, 2 * TAILROWS), jnp.float32),
        ],
        compiler_params=pltpu.CompilerParams(
            use_tc_tiling_on_sc=True, needs_layout_passes=False),
    )(tT)


def _body(idxT_hbm, packed_hbm, outT_hbm, idx_v,
          qv0, jv0, buf0, oblk0, gsem0,
          qv1, jv1, buf1, oblk1, gsem1):
    nc = 2
    wid = lax.axis_index("s") * nc + lax.axis_index("c")
    s0 = wid * SLAB
    pltpu.sync_copy(idxT_hbm.at[:, pl.ds(s0, SLAB)], idx_v)

    def prep_start(b, qv, jv, buf, gsem):
        p = lax.div(b, BPS)
        sb = lax.rem(b, BPS)

        @pl.loop(0, SB // 16)
        def _q(k):
            v = idx_v[p, pl.ds(sb * SB + k * 16, 16)]
            qv[pl.ds(k * 16, 16)] = lax.shift_right_logical(v, 1)
            jv[pl.ds(k * 16, 16)] = lax.bitwise_and(v, 1)

        pltpu.async_copy(packed_hbm.at[qv], buf, gsem)

    def finish(b, qv, jv, buf, oblk, gsem):
        p = lax.div(b, BPS)
        sb = lax.rem(b, BPS)
        pltpu.make_async_copy(packed_hbm.at[qv], buf, gsem).wait()

        @plsc.parallel_loop(0, SB // 16, unroll=4)
        def _tb(tb):
            i16 = lax.iota(jnp.int32, 16)
            trow = i16 + tb * 16
            cbase = jv[pl.ds(tb * 16, 16)] * 64
            for db in range(DIM // 16):
                cb = cbase + db * 16
                for k in range(16):
                    dd = lax.bitwise_and(i16 + k, 15)
                    vals = plsc.load_gather(buf, [trow, cb + dd])
                    plsc.store_scatter(oblk, [db * 16 + dd, trow], vals)

        pltpu.sync_copy(oblk, outT_hbm.at[p, :, pl.ds(s0 + sb * SB, SB)])

    prep_start(0, qv0, jv0, buf0, gsem0)

    @pl.loop(0, NBLK, step=2)
    def _pair(g):
        prep_start(g + 1, qv1, jv1, buf1, gsem1)
        finish(g, qv0, jv0, buf0, oblk0, gsem0)

        @pl.when(g + 2 < NBLK)
        def _pre():
            prep_start(g + 2, qv0, jv0, buf0, gsem0)

        finish(g + 1, qv1, jv1, buf1, oblk1, gsem1)


@jax.jit
def _gather2(idxT, packed):
    mesh = plsc.VectorSubcoreMesh(core_axis_name="c", subcore_axis_name="s")
    return pl.kernel(
        _body,
        out_type=jax.ShapeDtypeStruct((SEQ, DIM, NSEQ), jnp.float32),
        mesh=mesh,
        scratch_types=[
            pltpu.VMEM((SEQ, SLAB), jnp.int32),
        ] + 2 * [
            pltpu.VMEM((SB,), jnp.int32),
            pltpu.VMEM((SB,), jnp.int32),
            pltpu.VMEM((SB, 128), jnp.float32),
            pltpu.VMEM((DIM, SB), jnp.float32),
            pltpu.SemaphoreType.DMA,
        ],
        compiler_params=pltpu.CompilerParams(
            use_tc_tiling_on_sc=True, needs_layout_passes=False),
    )(idxT, packed)


def kernel(indices, table):
    packed = _repack(table.T)
    outT = _gather2(indices.T, packed)
    return jnp.transpose(outT, (2, 0, 1))
